# baseline (speedup 1.0000x reference)
import functools
import math

import jax
import jax.numpy as jnp
from jax import lax
from jax.experimental import pallas as pl
from jax.experimental.pallas import tpu as pltpu
from jax.experimental.pallas import tpu_sc as plsc

D_MODEL = 32
SCALE = math.sqrt(D_MODEL)

NC = 2
NS = 16
NW = NC * NS
LANES = 16

CHUNK = 1600
NBUF = 2


def _make_gather(S0: int, S1: int):
    B = S0 * S1
    assert B % NW == 0 and CHUNK % S1 == 0
    per_w = B // NW
    assert per_w % (CHUNK * NBUF) == 0
    n_outer = per_w // (CHUNK * NBUF)
    rows_per_chunk = CHUNK // S1
    s0_per_w = per_w // S1

    mesh = plsc.VectorSubcoreMesh(core_axis_name="c", subcore_axis_name="s")

    @functools.partial(
        pl.kernel,
        mesh=mesh,
        compiler_params=pltpu.CompilerParams(use_tc_tiling_on_sc=False),
        out_type=jax.ShapeDtypeStruct((S0, S1, D_MODEL), jnp.float32),
        scratch_types=[
            pltpu.VMEM((NBUF, CHUNK), jnp.int32),
            pltpu.VMEM((NBUF, CHUNK, D_MODEL), jnp.float32),
            [pltpu.SemaphoreType.DMA] * NBUF,
            [pltpu.SemaphoreType.DMA] * NBUF,
        ],
    )
    def gather_kernel(table_hbm, idx_hbm, out_hbm, idx_v, rows_v, gsems, osems):
        wid = lax.axis_index("s") * NC + lax.axis_index("c")
        base = wid * per_w
        s0_base = wid * s0_per_w

        def gather_copy(b):
            return pltpu.make_async_copy(
                table_hbm.at[idx_v.at[b]], rows_v.at[b], gsems[b]
            )

        for b in range(NBUF):
            pltpu.sync_copy(idx_hbm.at[pl.ds(base + b * CHUNK, CHUNK)], idx_v.at[b])
            gather_copy(b).start()

        def outer(o, carry):
            for b in range(NBUF):
                g = o * NBUF + b
                off = base + g * CHUNK
                s0_off = s0_base + g * rows_per_chunk
                gather_copy(b).wait()

                def scale_body(r, c):
                    lo = rows_v[b, r, pl.ds(0, LANES)]
                    rows_v[b, r, pl.ds(0, LANES)] = lo * SCALE
                    hi = rows_v[b, r, pl.ds(LANES, LANES)]
                    rows_v[b, r, pl.ds(LANES, LANES)] = hi * SCALE
                    return c

                lax.fori_loop(0, CHUNK, scale_body, 0, unroll=8)

                out_copies = [
                    pltpu.make_async_copy(
                        rows_v.at[b, pl.ds(k * S1, S1)],
                        out_hbm.at[s0_off + k],
                        osems[b],
                    )
                    for k in range(rows_per_chunk)
                ]
                for c in out_copies:
                    c.start()

                not_last = o < n_outer - 1

                @pl.when(not_last)
                def _prefetch_idx():
                    pltpu.sync_copy(
                        idx_hbm.at[pl.ds(off + NBUF * CHUNK, CHUNK)], idx_v.at[b]
                    )

                for c in out_copies:
                    c.wait()

                @pl.when(not_last)
                def _launch_gather():
                    gather_copy(b).start()

            return carry

        lax.fori_loop(0, n_outer, outer, 0)

    return gather_kernel


SLICES = (512,) + (1536,) * 10 + (512,)


def kernel(x, table):
    S0, S1 = x.shape
    assert sum(SLICES) == S0
    gathers = {s: _make_gather(s, S1) for s in set(SLICES)}
    parts = []
    row = 0
    for s in SLICES:
        idx_k = x[row:row + s].reshape(s * S1).astype(jnp.int32)
        parts.append(gathers[s](table, idx_k))
        row += s
    return jnp.concatenate(parts, axis=0)


# speedup vs baseline: 1.0134x; 1.0134x over previous
import functools
import math

import jax
import jax.numpy as jnp
from jax import lax
from jax.experimental import pallas as pl
from jax.experimental.pallas import tpu as pltpu
from jax.experimental.pallas import tpu_sc as plsc

D_MODEL = 32
SCALE = math.sqrt(D_MODEL)

NC = 2
NS = 16
NW = NC * NS
LANES = 16

CHUNK = 800
NBUF = 2


def _make_gather(S0: int, S1: int):
    B = S0 * S1
    assert B % NW == 0 and CHUNK % S1 == 0
    per_w = B // NW
    assert per_w % (CHUNK * NBUF) == 0
    n_outer = per_w // (CHUNK * NBUF)
    rows_per_chunk = CHUNK // S1
    s0_per_w = per_w // S1

    mesh = plsc.VectorSubcoreMesh(core_axis_name="c", subcore_axis_name="s")

    @functools.partial(
        pl.kernel,
        mesh=mesh,
        compiler_params=pltpu.CompilerParams(use_tc_tiling_on_sc=False),
        out_type=jax.ShapeDtypeStruct((S0, S1, D_MODEL), jnp.float32),
        scratch_types=[
            pltpu.VMEM((NBUF, CHUNK), jnp.int32),
            pltpu.VMEM((NBUF, CHUNK, D_MODEL), jnp.float32),
            [pltpu.SemaphoreType.DMA] * NBUF,
            [pltpu.SemaphoreType.DMA] * NBUF,
        ],
    )
    def gather_kernel(table_hbm, idx_hbm, out_hbm, idx_v, rows_v, gsems, osems):
        wid = lax.axis_index("s") * NC + lax.axis_index("c")
        base = wid * per_w
        s0_base = wid * s0_per_w

        def gather_copy(b):
            return pltpu.make_async_copy(
                table_hbm.at[idx_v.at[b]], rows_v.at[b], gsems[b]
            )

        for b in range(NBUF):
            pltpu.sync_copy(idx_hbm.at[pl.ds(base + b * CHUNK, CHUNK)], idx_v.at[b])
            gather_copy(b).start()

        def outer(o, carry):
            for b in range(NBUF):
                g = o * NBUF + b
                off = base + g * CHUNK
                s0_off = s0_base + g * rows_per_chunk
                gather_copy(b).wait()

                def scale_body(r, c):
                    lo = rows_v[b, r, pl.ds(0, LANES)]
                    rows_v[b, r, pl.ds(0, LANES)] = lo * SCALE
                    hi = rows_v[b, r, pl.ds(LANES, LANES)]
                    rows_v[b, r, pl.ds(LANES, LANES)] = hi * SCALE
                    return c

                lax.fori_loop(0, CHUNK, scale_body, 0, unroll=8)

                out_copies = [
                    pltpu.make_async_copy(
                        rows_v.at[b, pl.ds(k * S1, S1)],
                        out_hbm.at[s0_off + k],
                        osems[b],
                    )
                    for k in range(rows_per_chunk)
                ]
                for c in out_copies:
                    c.start()

                not_last = o < n_outer - 1

                @pl.when(not_last)
                def _prefetch_idx():
                    pltpu.sync_copy(
                        idx_hbm.at[pl.ds(off + NBUF * CHUNK, CHUNK)], idx_v.at[b]
                    )

                for c in out_copies:
                    c.wait()

                @pl.when(not_last)
                def _launch_gather():
                    gather_copy(b).start()

            return carry

        lax.fori_loop(0, n_outer, outer, 0)

    return gather_kernel


NSLICE = 8


def kernel(x, table):
    S0, S1 = x.shape
    Sk = S0 // NSLICE
    gather = _make_gather(Sk, S1)
    parts = []
    for k in range(NSLICE):
        idx_k = x[k * Sk:(k + 1) * Sk].reshape(Sk * S1).astype(jnp.int32)
        parts.append(gather(table, idx_k))
    return jnp.concatenate(parts, axis=0)


# speedup vs baseline: 1.0152x; 1.0018x over previous
import functools
import math

import jax
import jax.numpy as jnp
from jax import lax
from jax.experimental import pallas as pl
from jax.experimental.pallas import tpu as pltpu
from jax.experimental.pallas import tpu_sc as plsc

D_MODEL = 32
SCALE = math.sqrt(D_MODEL)

NC = 2
NS = 16
NW = NC * NS
LANES = 16

CHUNK = 1600
NBUF = 2


def _make_gather(S0: int, S1: int):
    B = S0 * S1
    assert B % NW == 0 and CHUNK % S1 == 0
    per_w = B // NW
    assert per_w % (CHUNK * NBUF) == 0
    n_outer = per_w // (CHUNK * NBUF)
    rows_per_chunk = CHUNK // S1
    s0_per_w = per_w // S1

    mesh = plsc.VectorSubcoreMesh(core_axis_name="c", subcore_axis_name="s")

    @functools.partial(
        pl.kernel,
        mesh=mesh,
        compiler_params=pltpu.CompilerParams(use_tc_tiling_on_sc=False),
        out_type=jax.ShapeDtypeStruct((S0, S1, D_MODEL), jnp.float32),
        scratch_types=[
            pltpu.VMEM((NBUF, CHUNK), jnp.int32),
            pltpu.VMEM((NBUF, CHUNK, D_MODEL), jnp.float32),
            [pltpu.SemaphoreType.DMA] * NBUF,
            [pltpu.SemaphoreType.DMA] * NBUF,
        ],
    )
    def gather_kernel(table_hbm, idx_hbm, out_hbm, idx_v, rows_v, gsems, osems):
        wid = lax.axis_index("s") * NC + lax.axis_index("c")
        base = wid * per_w
        s0_base = wid * s0_per_w

        def gather_copy(b):
            return pltpu.make_async_copy(
                table_hbm.at[idx_v.at[b]], rows_v.at[b], gsems[b]
            )

        for b in range(NBUF):
            pltpu.sync_copy(idx_hbm.at[pl.ds(base + b * CHUNK, CHUNK)], idx_v.at[b])
            gather_copy(b).start()

        def outer(o, carry):
            for b in range(NBUF):
                g = o * NBUF + b
                off = base + g * CHUNK
                s0_off = s0_base + g * rows_per_chunk
                gather_copy(b).wait()

                def scale_body(r, c):
                    lo = rows_v[b, r, pl.ds(0, LANES)]
                    rows_v[b, r, pl.ds(0, LANES)] = lo * SCALE
                    hi = rows_v[b, r, pl.ds(LANES, LANES)]
                    rows_v[b, r, pl.ds(LANES, LANES)] = hi * SCALE
                    return c

                lax.fori_loop(0, CHUNK, scale_body, 0, unroll=8)

                out_copies = [
                    pltpu.make_async_copy(
                        rows_v.at[b, pl.ds(k * S1, S1)],
                        out_hbm.at[s0_off + k],
                        osems[b],
                    )
                    for k in range(rows_per_chunk)
                ]
                for c in out_copies:
                    c.start()

                not_last = o < n_outer - 1

                @pl.when(not_last)
                def _prefetch_idx():
                    pltpu.sync_copy(
                        idx_hbm.at[pl.ds(off + NBUF * CHUNK, CHUNK)], idx_v.at[b]
                    )

                for c in out_copies:
                    c.wait()

                @pl.when(not_last)
                def _launch_gather():
                    gather_copy(b).start()

            return carry

        lax.fori_loop(0, n_outer, outer, 0)

    return gather_kernel


NSLICE = 8


def kernel(x, table):
    S0, S1 = x.shape
    Sk = S0 // NSLICE
    gather = _make_gather(Sk, S1)
    parts = []
    for k in range(NSLICE):
        idx_k = x[k * Sk:(k + 1) * Sk].reshape(Sk * S1).astype(jnp.int32)
        parts.append(gather(table, idx_k))
    return jnp.concatenate(parts, axis=0)
